# baseline (speedup 1.0000x reference)
import jax
import jax.numpy as jnp
from jax.experimental import pallas as pl
from jax.experimental.pallas import tpu as pltpu

_NEG = -1e30


def _body(cats_ref, labels_ref, x_ref, z_ref, out_ref, acc_ref):
    b = pl.program_id(0)
    nb = pl.num_programs(0)
    cat = cats_ref[b]
    x = x_ref[0]
    z = z_ref[0]
    m = z == cat
    part = jnp.max(jnp.where(m, x, _NEG))

    valid = (cat > 0) & (part > -9e29)
    r = jnp.where(valid, part, 0.0)
    y = labels_ref[b]
    per = jnp.maximum(r, 0.0) - r * y + jnp.log1p(jnp.exp(-jnp.abs(r)))

    @pl.when(b == 0)
    def _init():
        acc_ref[0] = per

    @pl.when(b > 0)
    def _acc():
        acc_ref[0] = acc_ref[0] + per

    @pl.when(b == nb - 1)
    def _emit():
        out_ref[0, 0] = acc_ref[0] / nb


def kernel(pixel_logits, zone_patches, cats, labels):
    B, _, H, W = pixel_logits.shape
    logits = pixel_logits.reshape(B, H, W)

    grid_spec = pltpu.PrefetchScalarGridSpec(
        num_scalar_prefetch=2,
        grid=(B,),
        in_specs=[
            pl.BlockSpec((1, H, W), lambda b, cats, labels: (b, 0, 0)),
            pl.BlockSpec((1, H, W), lambda b, cats, labels: (b, 0, 0)),
        ],
        out_specs=pl.BlockSpec(memory_space=pltpu.SMEM),
        scratch_shapes=[pltpu.SMEM((1,), jnp.float32)],
    )
    loss = pl.pallas_call(
        _body,
        grid_spec=grid_spec,
        out_shape=jax.ShapeDtypeStruct((1, 1), jnp.float32),
    )(cats, labels, logits, zone_patches)

    return loss[0, 0]


# speedup vs baseline: 1.7288x; 1.7288x over previous
import functools

import jax
import jax.numpy as jnp
from jax.experimental import pallas as pl
from jax.experimental.pallas import tpu as pltpu

_NEG = -1e30
_NSLOT = 4


def _body(cats_ref, labels_ref, x_hbm, z_hbm, out_ref, xbuf, zbuf, acc_ref,
          xsem, zsem):
    B = x_hbm.shape[0]

    def start(b, slot):
        pltpu.make_async_copy(x_hbm.at[b], xbuf.at[slot], xsem.at[slot]).start()
        pltpu.make_async_copy(z_hbm.at[b], zbuf.at[slot], zsem.at[slot]).start()

    for b in range(_NSLOT):
        start(b, b)

    def step(b, loss_sum):
        slot = jax.lax.rem(b, _NSLOT)
        pltpu.make_async_copy(x_hbm.at[0], xbuf.at[slot], xsem.at[slot]).wait()
        pltpu.make_async_copy(z_hbm.at[0], zbuf.at[slot], zsem.at[slot]).wait()
        x = xbuf[slot]
        z = zbuf[slot]
        cat = cats_ref[b]
        part = jnp.max(jnp.where(z == cat, x, _NEG))

        @pl.when(b + _NSLOT < B)
        def _next():
            start(b + _NSLOT, slot)

        valid = (cat > 0) & (part > -9e29)
        r = jnp.where(valid, part, 0.0)
        y = labels_ref[b]
        per = jnp.maximum(r, 0.0) - r * y + jnp.log1p(jnp.exp(-jnp.abs(r)))
        return loss_sum + per

    loss_sum = jax.lax.fori_loop(0, B, step, jnp.float32(0.0))
    out_ref[0, 0] = loss_sum / B


def kernel(pixel_logits, zone_patches, cats, labels):
    B, _, H, W = pixel_logits.shape
    logits = pixel_logits.reshape(B, H, W)

    grid_spec = pltpu.PrefetchScalarGridSpec(
        num_scalar_prefetch=2,
        grid=(),
        in_specs=[
            pl.BlockSpec(memory_space=pl.ANY),
            pl.BlockSpec(memory_space=pl.ANY),
        ],
        out_specs=pl.BlockSpec(memory_space=pltpu.SMEM),
        scratch_shapes=[
            pltpu.VMEM((_NSLOT, H, W), jnp.float32),
            pltpu.VMEM((_NSLOT, H, W), jnp.int32),
            pltpu.SMEM((1,), jnp.float32),
            pltpu.SemaphoreType.DMA((_NSLOT,)),
            pltpu.SemaphoreType.DMA((_NSLOT,)),
        ],
    )
    loss = pl.pallas_call(
        _body,
        grid_spec=grid_spec,
        out_shape=jax.ShapeDtypeStruct((1, 1), jnp.float32),
    )(cats, labels, logits, zone_patches)

    return loss[0, 0]
